# baseline (speedup 1.0000x reference)
import functools
import jax
import jax.numpy as jnp
from jax import lax
from jax.experimental import pallas as pl
from jax.experimental.pallas import tpu as pltpu
from jax.experimental.pallas import tpu_sc as plsc

B, N, F = 16, 4096, 512
E = 64
HI = 2048
HALF = N // 2
TE = E // 2
LANES = 16
CPR = F // LANES
C = 32
NCHUNK = HALF // C
NBUF = 4
NGRP = NCHUNK // NBUF

_mesh = plsc.VectorSubcoreMesh(core_axis_name="c", subcore_axis_name="s")


@functools.partial(
    pl.kernel,
    mesh=_mesh,
    out_type=jax.ShapeDtypeStruct((B, N + E, F), jnp.float32),
    scratch_types=[
        pltpu.VMEM((NBUF, C, F), jnp.float32),
        pltpu.VMEM((TE, F), jnp.float32),
        pltpu.VMEM((TE, F), jnp.float32),
        pltpu.SemaphoreType.DMA,
        pltpu.SemaphoreType.DMA,
        pltpu.SemaphoreType.DMA,
        pltpu.SemaphoreType.DMA,
        pltpu.SemaphoreType.DMA,
        pltpu.SemaphoreType.DMA,
        pltpu.SemaphoreType.DMA,
        pltpu.SemaphoreType.DMA,
        pltpu.SemaphoreType.DMA,
    ],
)
def _sc_unpool(x_hbm, out_hbm, bufs, lo_v, hi_v,
               in0, in1, in2, in3, out0, out1, out2, out3, tail_sem):
    cid = lax.axis_index("c")
    sid = lax.axis_index("s")
    b = sid
    h = cid
    r0 = h * HALF
    t0 = h * TE

    in_sems = (in0, in1, in2, in3)
    out_sems = (out0, out1, out2, out3)

    def in_copy(i, par):
        return pltpu.make_async_copy(
            x_hbm.at[b, pl.ds(r0 + i * C, C), :], bufs.at[par], in_sems[par]
        )

    def out_copy(i, par):
        return pltpu.make_async_copy(
            bufs.at[par], out_hbm.at[b, pl.ds(r0 + i * C, C), :], out_sems[par]
        )

    ld_lo = pltpu.make_async_copy(x_hbm.at[b, pl.ds(t0, TE), :], lo_v, tail_sem)
    ld_hi = pltpu.make_async_copy(x_hbm.at[b, pl.ds(HI + t0, TE), :], hi_v, tail_sem)
    ld_lo.start()
    ld_hi.start()

    for p in range(NBUF):
        in_copy(p, p).start()

    def outer(i, carry):
        for p in range(NBUF):
            in_copy(i + p, p).wait()
            out_copy(i + p, p).start()
        for p in range(NBUF):
            @pl.when(i + NBUF + p < NCHUNK)
            def _prefetch(p=p):
                out_copy(i + p, p).wait()
                in_copy(i + NBUF + p, p).start()
        return carry

    lax.fori_loop(0, NGRP, lambda k, c2: outer(k * NBUF, c2), 0)

    for p in range(NBUF):
        out_copy(NCHUNK - NBUF + p, p).wait()

    ld_lo.wait()
    ld_hi.wait()

    def _row(r, carry):
        for ci in range(CPR):
            cc = ci * LANES
            lo_v[r, pl.ds(cc, LANES)] = 0.5 * (
                lo_v[r, pl.ds(cc, LANES)] + hi_v[r, pl.ds(cc, LANES)]
            )
        return carry

    lax.fori_loop(0, TE, _row, 0)

    st = pltpu.make_async_copy(lo_v, out_hbm.at[b, pl.ds(N + t0, TE), :], tail_sem)
    st.start()
    st.wait()


def kernel(inputs):
    return _sc_unpool(inputs)


# speedup vs baseline: 1.0666x; 1.0666x over previous
import functools
import jax
import jax.numpy as jnp
from jax import lax
from jax.experimental import pallas as pl
from jax.experimental.pallas import tpu as pltpu
from jax.experimental.pallas import tpu_sc as plsc

B, N, F = 16, 4096, 512
E = 64
HI = 2048
TE = E // 2
LANES = 16
CPR = F // LANES

_mesh = plsc.VectorSubcoreMesh(core_axis_name="c", subcore_axis_name="s")


@functools.partial(
    pl.kernel,
    mesh=_mesh,
    out_type=jax.ShapeDtypeStruct((B, E, F), jnp.float32),
    scratch_types=[
        pltpu.VMEM((TE, F), jnp.float32),
        pltpu.VMEM((TE, F), jnp.float32),
        pltpu.SemaphoreType.DMA,
    ],
)
def _sc_midpoints(x_hbm, nv_hbm, lo_v, hi_v, sem):
    cid = lax.axis_index("c")
    sid = lax.axis_index("s")
    b = sid
    t0 = cid * TE

    ld_lo = pltpu.make_async_copy(x_hbm.at[b, pl.ds(t0, TE), :], lo_v, sem)
    ld_hi = pltpu.make_async_copy(x_hbm.at[b, pl.ds(HI + t0, TE), :], hi_v, sem)
    ld_lo.start()
    ld_hi.start()
    ld_lo.wait()
    ld_hi.wait()

    def _row(r, carry):
        for ci in range(CPR):
            cc = ci * LANES
            lo_v[r, pl.ds(cc, LANES)] = 0.5 * (
                lo_v[r, pl.ds(cc, LANES)] + hi_v[r, pl.ds(cc, LANES)]
            )
        return carry

    lax.fori_loop(0, TE, _row, 0)

    st = pltpu.make_async_copy(lo_v, nv_hbm.at[b, pl.ds(t0, TE), :], sem)
    st.start()
    st.wait()


RB = 2080
NBLK = (N + E) // RB
TAIL_COPY = N - (NBLK - 1) * RB


def _tc_body(x_ref, nv_ref, out_ref):
    j = pl.program_id(1)

    @pl.when(j < NBLK - 1)
    def _copy():
        out_ref[...] = x_ref[...]

    @pl.when(j == NBLK - 1)
    def _tail():
        out_ref[0, :TAIL_COPY, :] = x_ref[0, :TAIL_COPY, :]
        out_ref[0, TAIL_COPY:, :] = nv_ref[0]


def kernel(inputs):
    nv = _sc_midpoints(inputs)
    return pl.pallas_call(
        _tc_body,
        grid=(B, NBLK),
        in_specs=[
            pl.BlockSpec((1, RB, F), lambda b, j: (b, j, 0)),
            pl.BlockSpec((1, E, F), lambda b, j: (b, 0, 0)),
        ],
        out_specs=pl.BlockSpec((1, RB, F), lambda b, j: (b, j, 0)),
        out_shape=jax.ShapeDtypeStruct((B, N + E, F), inputs.dtype),
    )(inputs, nv)
